# Initial kernel scaffold; baseline (speedup 1.0000x reference)
import jax
import jax.numpy as jnp
from jax.experimental import pallas as pl


def kernel(x, adjacency, weight):
    raise NotImplementedError("write your pallas kernel here")


# speedup vs baseline: 1.6201x; 1.6201x over previous
import functools

import jax
import jax.numpy as jnp
from jax.experimental import pallas as pl
from jax.experimental.pallas import tpu as pltpu

N = 10000
D = 128
BM = 400
NI = N // BM
XE_W = 256


def _body(a_ref, xe_ref, w_ref, o_ref):
    acc = jax.lax.dot_general(
        a_ref[...], xe_ref[...], (((1,), (0,)), ((), ())),
        preferred_element_type=jnp.float32,
    )
    agg = acc[:, :D]
    rowsum = acc[:, D:D + 1]
    out = jax.lax.dot_general(
        agg, w_ref[...], (((1,), (0,)), ((), ())),
        preferred_element_type=jnp.float32,
    )
    o_ref[...] = out / rowsum


@functools.partial(jax.jit, static_argnames=("interpret",))
def _sageconv(x, adjacency, weight, interpret=False):
    xe = jnp.zeros((N, XE_W), dtype=jnp.float32)
    xe = xe.at[:, :D].set(x)
    xe = xe.at[:, D].set(1.0)

    return pl.pallas_call(
        _body,
        grid=(NI,),
        in_specs=[
            pl.BlockSpec((BM, N), lambda i: (i, 0)),
            pl.BlockSpec((N, XE_W), lambda i: (0, 0)),
            pl.BlockSpec((D, D), lambda i: (0, 0)),
        ],
        out_specs=pl.BlockSpec((BM, D), lambda i: (i, 0)),
        out_shape=jax.ShapeDtypeStruct((N, D), jnp.float32),
        compiler_params=pltpu.CompilerParams(
            dimension_semantics=("arbitrary",),
        ),
        interpret=interpret,
    )(adjacency, xe, weight)


def kernel(x, adjacency, weight):
    return _sageconv(x, adjacency, weight)


# speedup vs baseline: 1.6515x; 1.0194x over previous
import functools

import jax
import jax.numpy as jnp
from jax.experimental import pallas as pl
from jax.experimental.pallas import tpu as pltpu

N = 10000
D = 128
BM = 400
NI = N // BM
XE_W = 256


def _body(a_ref, xe_ref, w_ref, o_ref):
    acc = jax.lax.dot_general(
        a_ref[...].astype(jnp.bfloat16), xe_ref[...], (((1,), (0,)), ((), ())),
        preferred_element_type=jnp.float32,
    )
    agg = acc[:, :D]
    rowsum = acc[:, D:D + 1]
    out = jax.lax.dot_general(
        agg, w_ref[...], (((1,), (0,)), ((), ())),
        preferred_element_type=jnp.float32,
    )
    o_ref[...] = out / rowsum


@functools.partial(jax.jit, static_argnames=("interpret",))
def _sageconv(x, adjacency, weight, interpret=False):
    xe = jnp.zeros((N, XE_W), dtype=jnp.float32)
    xe = xe.at[:, :D].set(x)
    xe = xe.at[:, D].set(1.0)
    xe = xe.astype(jnp.bfloat16)

    return pl.pallas_call(
        _body,
        grid=(NI,),
        in_specs=[
            pl.BlockSpec((BM, N), lambda i: (i, 0)),
            pl.BlockSpec((N, XE_W), lambda i: (0, 0)),
            pl.BlockSpec((D, D), lambda i: (0, 0)),
        ],
        out_specs=pl.BlockSpec((BM, D), lambda i: (i, 0)),
        out_shape=jax.ShapeDtypeStruct((N, D), jnp.float32),
        compiler_params=pltpu.CompilerParams(
            dimension_semantics=("arbitrary",),
        ),
        interpret=interpret,
    )(adjacency, xe, weight)


def kernel(x, adjacency, weight):
    return _sageconv(x, adjacency, weight)
